# baseline (speedup 1.0000x reference)
import jax
import jax.numpy as jnp
from jax import lax
from jax.experimental import pallas as pl
from jax.experimental.pallas import tpu as pltpu
from jax.experimental.pallas import tpu_sc as plsc

N_NODES = 10000
NPAD = 10240
N_EDGES = 320000
F = 64
F2 = 48
K = 3
NEG_SLOPE = 0.2
EPS = 1e-5

NC = 2
NS = 16
NW = NC * NS
EPW = N_EDGES // NW
CHE = 125
NCH = EPW // CHE
RPS = NPAD // NS
NBUF = 8

_MESH = plsc.VectorSubcoreMesh(core_axis_name="c", subcore_axis_name="s")
_SC_PARAMS = pltpu.CompilerParams(use_tc_tiling_on_sc=False)


def _hop_body(v_hbm, srcr, dstr, zeros_hbm, out_hbm, src_i, dst_i, bufs,
              gsem, ssem, acc):
    c = lax.axis_index("c")
    s = lax.axis_index("s")
    wid = c * NS + s
    pltpu.sync_copy(srcr.at[wid], src_i)
    pltpu.sync_copy(dstr.at[wid], dst_i)
    pltpu.sync_copy(zeros_hbm.at[pl.ds(s * RPS, RPS)],
                    acc.at[pl.ds(s * RPS, RPS)])
    plsc.subcore_barrier()

    for b in range(NBUF):
        pltpu.async_copy(v_hbm.at[src_i.at[b]], bufs.at[b], gsem.at[b])

    @pl.loop(0, NCH - NBUF, step=NBUF)
    def _outer(j0):
        for b in range(NBUF):
            pltpu.make_async_copy(v_hbm.at[src_i.at[0]], bufs.at[b],
                                  gsem.at[b]).wait()
            pltpu.async_copy(bufs.at[b], acc.at[dst_i.at[j0 + b]],
                             ssem.at[b], add=True)
        for b in range(NBUF):
            pltpu.make_async_copy(bufs.at[b], acc.at[dst_i.at[0]],
                                  ssem.at[b]).wait()
            pltpu.async_copy(v_hbm.at[src_i.at[j0 + NBUF + b]], bufs.at[b],
                             gsem.at[b])

    for b in range(NBUF):
        pltpu.make_async_copy(v_hbm.at[src_i.at[0]], bufs.at[b],
                              gsem.at[b]).wait()
        pltpu.async_copy(bufs.at[b], acc.at[dst_i.at[NCH - NBUF + b]],
                         ssem.at[b], add=True)
    for b in range(NBUF):
        pltpu.make_async_copy(bufs.at[b], acc.at[dst_i.at[0]],
                              ssem.at[b]).wait()

    plsc.subcore_barrier()
    pltpu.sync_copy(acc.at[pl.ds(s * RPS, RPS)],
                    out_hbm.at[c, pl.ds(s * RPS, RPS)])


def _make_hop(f):
    return pl.kernel(
        _hop_body,
        out_type=jax.ShapeDtypeStruct((NC, NPAD, f), jnp.float32),
        mesh=_MESH,
        compiler_params=_SC_PARAMS,
        scratch_types=[
            pltpu.VMEM((NCH, CHE), jnp.int32),
            pltpu.VMEM((NCH, CHE), jnp.int32),
            pltpu.VMEM((NBUF, CHE, f), jnp.float32),
            pltpu.SemaphoreType.DMA((NBUF,)),
            pltpu.SemaphoreType.DMA((NBUF,)),
            pltpu.VMEM_SHARED((NPAD, f), jnp.float32),
        ],
    )


_hop_call = {f: _make_hop(f) for f in (F, F2)}


def _deg_body(srcr, dstr, ones_hbm, zeros_hbm, dego_hbm, degi_hbm,
              src_i, dst_i, ones_b, osem, isem, acco, acci):
    c = lax.axis_index("c")
    s = lax.axis_index("s")
    wid = c * NS + s
    pltpu.sync_copy(srcr.at[wid], src_i)
    pltpu.sync_copy(dstr.at[wid], dst_i)
    pltpu.sync_copy(ones_hbm, ones_b)
    pltpu.sync_copy(zeros_hbm.at[pl.ds(s * RPS, RPS)],
                    acco.at[pl.ds(s * RPS, RPS)])
    pltpu.sync_copy(zeros_hbm.at[pl.ds(s * RPS, RPS)],
                    acci.at[pl.ds(s * RPS, RPS)])
    plsc.subcore_barrier()

    for b in range(NBUF):
        pltpu.async_copy(ones_b, acco.at[src_i.at[b]], osem.at[b], add=True)
        pltpu.async_copy(ones_b, acci.at[dst_i.at[b]], isem.at[b], add=True)

    @pl.loop(NBUF, NCH, step=NBUF)
    def _step(j0):
        for b in range(NBUF):
            pltpu.make_async_copy(ones_b, acco.at[src_i.at[0]], osem.at[b]).wait()
            pltpu.make_async_copy(ones_b, acci.at[dst_i.at[0]], isem.at[b]).wait()
            pltpu.async_copy(ones_b, acco.at[src_i.at[j0 + b]], osem.at[b], add=True)
            pltpu.async_copy(ones_b, acci.at[dst_i.at[j0 + b]], isem.at[b], add=True)

    for b in range(NBUF):
        pltpu.make_async_copy(ones_b, acco.at[src_i.at[0]], osem.at[b]).wait()
        pltpu.make_async_copy(ones_b, acci.at[dst_i.at[0]], isem.at[b]).wait()

    plsc.subcore_barrier()
    pltpu.sync_copy(acco.at[pl.ds(s * RPS, RPS)],
                    dego_hbm.at[c, pl.ds(s * RPS, RPS)])
    pltpu.sync_copy(acci.at[pl.ds(s * RPS, RPS)],
                    degi_hbm.at[c, pl.ds(s * RPS, RPS)])


_deg_call = pl.kernel(
    _deg_body,
    out_type=(jax.ShapeDtypeStruct((NC, NPAD, 16), jnp.float32),
              jax.ShapeDtypeStruct((NC, NPAD, 16), jnp.float32)),
    mesh=_MESH,
    compiler_params=_SC_PARAMS,
    scratch_types=[
        pltpu.VMEM((NCH, CHE), jnp.int32),
        pltpu.VMEM((NCH, CHE), jnp.int32),
        pltpu.VMEM((CHE, 16), jnp.float32),
        pltpu.SemaphoreType.DMA((NBUF,)),
        pltpu.SemaphoreType.DMA((NBUF,)),
        pltpu.VMEM_SHARED((NPAD, 16), jnp.float32),
        pltpu.VMEM_SHARED((NPAD, 16), jnp.float32),
    ],
)


_BS = 512
_GRID = NPAD // _BS


def _norm_kernel(dego_ref, degi_ref, nsrc_ref, ndst_ref):
    dego = dego_ref[0, :, 0:1] + dego_ref[1, :, 0:1]
    degi = degi_ref[0, :, 0:1] + degi_ref[1, :, 0:1]
    nsrc_ref[...] = lax.rsqrt(jnp.maximum(dego, 1.0))
    ndst_ref[...] = lax.rsqrt(jnp.maximum(degi, 1.0))


def _norms(dego, degi):
    return pl.pallas_call(
        _norm_kernel,
        out_shape=(jax.ShapeDtypeStruct((NPAD, 1), jnp.float32),
                   jax.ShapeDtypeStruct((NPAD, 1), jnp.float32)),
    )(dego, degi)


def _mm_kernel(x_ref, w_ref, nsrc_ref, h_ref, v_ref):
    h = jnp.dot(x_ref[...], w_ref[...], preferred_element_type=jnp.float32)
    h_ref[...] = h
    v_ref[...] = h * nsrc_ref[...]


def _mm_prescale(x, w, nsrc):
    din = x.shape[1]
    f = w.shape[1]
    return pl.pallas_call(
        _mm_kernel,
        grid=(_GRID,),
        in_specs=[
            pl.BlockSpec((_BS, din), lambda i: (i, 0)),
            pl.BlockSpec((din, f), lambda i: (0, 0)),
            pl.BlockSpec((_BS, 1), lambda i: (i, 0)),
        ],
        out_specs=(pl.BlockSpec((_BS, f), lambda i: (i, 0)),
                   pl.BlockSpec((_BS, f), lambda i: (i, 0))),
        out_shape=(jax.ShapeDtypeStruct((NPAD, f), jnp.float32),
                   jax.ShapeDtypeStruct((NPAD, f), jnp.float32)),
    )(x, w, nsrc)


def _combine_kernel(p_ref, ndst_ref, nsrc_ref, cur_ref, v_ref):
    agg = p_ref[0] + p_ref[1]
    cur = agg * ndst_ref[...]
    cur_ref[...] = cur
    v_ref[...] = cur * nsrc_ref[...]


def _combine(parts, ndst, nsrc):
    f = parts.shape[2]
    return pl.pallas_call(
        _combine_kernel,
        grid=(_GRID,),
        in_specs=[
            pl.BlockSpec((NC, _BS, f), lambda i: (0, i, 0)),
            pl.BlockSpec((_BS, 1), lambda i: (i, 0)),
            pl.BlockSpec((_BS, 1), lambda i: (i, 0)),
        ],
        out_specs=(pl.BlockSpec((_BS, f), lambda i: (i, 0)),
                   pl.BlockSpec((_BS, f), lambda i: (i, 0))),
        out_shape=(jax.ShapeDtypeStruct((NPAD, f), jnp.float32),
                   jax.ShapeDtypeStruct((NPAD, f), jnp.float32)),
    )(parts, ndst, nsrc)


def _attn_kernel(h0_ref, h1_ref, h2_ref, h3_ref, pos_ref, al_ref, ar_ref,
                 b_ref, out_ref):
    hs = [h0_ref[...] + pos_ref[0:1, :],
          h1_ref[...] + pos_ref[1:2, :],
          h2_ref[...] + pos_ref[2:3, :],
          h3_ref[...] + pos_ref[3:4, :]]
    al = al_ref[...]
    ar = ar_ref[...]
    a_r = jnp.sum(hs[0] * ar, axis=1, keepdims=True)
    a = [jnp.sum(h * al, axis=1, keepdims=True) + a_r for h in hs]
    a = [jnp.where(ak >= 0, ak, NEG_SLOPE * ak) for ak in a]
    m = jnp.maximum(jnp.maximum(a[0], a[1]), jnp.maximum(a[2], a[3]))
    e = [jnp.exp(ak - m) for ak in a]
    denom = e[0] + e[1] + e[2] + e[3]
    rst = hs[0] * (e[0] / denom)
    for k2 in range(1, K + 1):
        rst = rst + hs[k2] * (e[k2] / denom)
    out_ref[...] = rst + b_ref[...]


def _attention(h0, h1, h2, h3, pos, al, ar, b):
    f = h0.shape[1]
    blk = lambda i: (i, 0)
    row = lambda i: (0, 0)
    return pl.pallas_call(
        _attn_kernel,
        grid=(_GRID,),
        in_specs=[
            pl.BlockSpec((_BS, f), blk),
            pl.BlockSpec((_BS, f), blk),
            pl.BlockSpec((_BS, f), blk),
            pl.BlockSpec((_BS, f), blk),
            pl.BlockSpec((K + 1, f), row),
            pl.BlockSpec((1, f), row),
            pl.BlockSpec((1, f), row),
            pl.BlockSpec((1, f), row),
        ],
        out_specs=pl.BlockSpec((_BS, f), blk),
        out_shape=jax.ShapeDtypeStruct((NPAD, f), jnp.float32),
    )(h0, h1, h2, h3, pos, al, ar, b)


def _bn_relu_kernel(x_ref, g_ref, be_ref, out_ref):
    x = x_ref[...]
    rows = lax.broadcasted_iota(jnp.int32, (NPAD, 1), 0)
    mask = rows < N_NODES
    xm = jnp.where(mask, x, 0.0)
    mean = jnp.sum(xm, axis=0, keepdims=True) / N_NODES
    d = x - mean
    dm = jnp.where(mask, d, 0.0)
    var = jnp.sum(dm * dm, axis=0, keepdims=True) / N_NODES
    y = d * lax.rsqrt(var + EPS) * g_ref[...] + be_ref[...]
    out_ref[...] = jnp.maximum(y, 0.0)


def _bn_relu(x, g, be):
    return pl.pallas_call(
        _bn_relu_kernel,
        out_shape=jax.ShapeDtypeStruct((NPAD, F), jnp.float32),
    )(x, g.reshape(1, F), be.reshape(1, F))


def _layer(h_in, srcr, dstr, zeros, nsrc, ndst, w, pos, al, ar, b):
    f = w.shape[1]
    h0, v = _mm_prescale(h_in, w, nsrc)
    hstack = [h0]
    for _ in range(K):
        parts = _hop_call[f](v, srcr, dstr, zeros)
        cur, v = _combine(parts, ndst, nsrc)
        hstack.append(cur)
    return _attention(hstack[0], hstack[1], hstack[2], hstack[3],
                      pos, al, ar, b)


@jax.jit
def kernel(x, edge_index, W0, pos0, al0, ar0, b0, g0, be0,
           W1, pos1, al1, ar1, b1, g1, be1, W2, pos2, al2, ar2, b2):
    f32 = jnp.float32
    xp = jnp.zeros((NPAD, x.shape[1]), f32).at[:N_NODES].set(x)
    srcr = edge_index[0].reshape(NW, NCH, CHE)
    dstr = edge_index[1].reshape(NW, NCH, CHE)
    zeros = jnp.zeros((NPAD, F), f32)
    zeros48 = jnp.zeros((NPAD, F2), f32)
    zeros16 = jnp.zeros((NPAD, 16), f32)
    ones16 = jnp.ones((CHE, 16), f32)

    dego, degi = _deg_call(srcr, dstr, ones16, zeros16)
    nsrc, ndst = _norms(dego, degi)

    w2p = jnp.zeros((F, F2), f32).at[:, :40].set(W2)
    pos2p = jnp.zeros((K + 1, F2), f32).at[:, :40].set(pos2.reshape(K + 1, 40))
    al2p = jnp.zeros((1, F2), f32).at[:, :40].set(al2.reshape(1, 40))
    ar2p = jnp.zeros((1, F2), f32).at[:, :40].set(ar2.reshape(1, 40))
    b2p = jnp.zeros((1, F2), f32).at[:, :40].set(b2.reshape(1, 40))

    h = _layer(xp, srcr, dstr, zeros, nsrc, ndst,
               W0, pos0.reshape(K + 1, F), al0.reshape(1, F),
               ar0.reshape(1, F), b0.reshape(1, F))
    h = _bn_relu(h, g0, be0)
    h = _layer(h, srcr, dstr, zeros, nsrc, ndst,
               W1, pos1.reshape(K + 1, F), al1.reshape(1, F),
               ar1.reshape(1, F), b1.reshape(1, F))
    h = _bn_relu(h, g1, be1)
    h = _layer(h, srcr, dstr, zeros48, nsrc, ndst,
               w2p, pos2p, al2p, ar2p, b2p)
    return h[:N_NODES, :40]


# speedup vs baseline: 1.0178x; 1.0178x over previous
import jax
import jax.numpy as jnp
from jax import lax
from jax.experimental import pallas as pl
from jax.experimental.pallas import tpu as pltpu
from jax.experimental.pallas import tpu_sc as plsc

N_NODES = 10000
NPAD = 10240
N_EDGES = 320000
F = 64
F2 = 48
K = 3
NEG_SLOPE = 0.2
EPS = 1e-5

NC = 2
NS = 16
NW = NC * NS
EPW = N_EDGES // NW
CHE = 125
NCH = EPW // CHE
RPS = NPAD // NS
NBUF = 8

_MESH = plsc.VectorSubcoreMesh(core_axis_name="c", subcore_axis_name="s")
_SC_PARAMS = pltpu.CompilerParams(use_tc_tiling_on_sc=False)


def _hop_body(v_hbm, srcr, dstr, zeros_hbm, out_hbm, src_i, dst_i, bufs,
              gsem, ssem, acc):
    c = lax.axis_index("c")
    s = lax.axis_index("s")
    wid = c * NS + s
    cp_src = pltpu.async_copy(srcr.at[wid], src_i, gsem.at[0])
    cp_dst = pltpu.async_copy(dstr.at[wid], dst_i, gsem.at[1])
    cp_z = pltpu.async_copy(zeros_hbm.at[pl.ds(s * RPS, RPS)],
                            acc.at[pl.ds(s * RPS, RPS)], gsem.at[2])
    cp_src.wait()
    cp_dst.wait()
    cp_z.wait()
    plsc.subcore_barrier()

    for b in range(NBUF):
        pltpu.async_copy(v_hbm.at[src_i.at[b]], bufs.at[b], gsem.at[b])

    @pl.loop(0, NCH - NBUF, step=NBUF)
    def _outer(j0):
        for b in range(NBUF):
            pltpu.make_async_copy(v_hbm.at[src_i.at[0]], bufs.at[b],
                                  gsem.at[b]).wait()
            pltpu.async_copy(bufs.at[b], acc.at[dst_i.at[j0 + b]],
                             ssem.at[b], add=True)
        for b in range(NBUF):
            pltpu.make_async_copy(bufs.at[b], acc.at[dst_i.at[0]],
                                  ssem.at[b]).wait()
            pltpu.async_copy(v_hbm.at[src_i.at[j0 + NBUF + b]], bufs.at[b],
                             gsem.at[b])

    for b in range(NBUF):
        pltpu.make_async_copy(v_hbm.at[src_i.at[0]], bufs.at[b],
                              gsem.at[b]).wait()
        pltpu.async_copy(bufs.at[b], acc.at[dst_i.at[NCH - NBUF + b]],
                         ssem.at[b], add=True)
    for b in range(NBUF):
        pltpu.make_async_copy(bufs.at[b], acc.at[dst_i.at[0]],
                              ssem.at[b]).wait()

    plsc.subcore_barrier()
    pltpu.sync_copy(acc.at[pl.ds(s * RPS, RPS)],
                    out_hbm.at[c, pl.ds(s * RPS, RPS)])


def _make_hop(f):
    return pl.kernel(
        _hop_body,
        out_type=jax.ShapeDtypeStruct((NC, NPAD, f), jnp.float32),
        mesh=_MESH,
        compiler_params=_SC_PARAMS,
        scratch_types=[
            pltpu.VMEM((NCH, CHE), jnp.int32),
            pltpu.VMEM((NCH, CHE), jnp.int32),
            pltpu.VMEM((NBUF, CHE, f), jnp.float32),
            pltpu.SemaphoreType.DMA((NBUF,)),
            pltpu.SemaphoreType.DMA((NBUF,)),
            pltpu.VMEM_SHARED((NPAD, f), jnp.float32),
        ],
    )


_hop_call = {f: _make_hop(f) for f in (F, F2)}


def _deg_body(srcr, dstr, ones_hbm, zeros_hbm, dego_hbm, degi_hbm,
              src_i, dst_i, ones_b, osem, isem, acco, acci):
    c = lax.axis_index("c")
    s = lax.axis_index("s")
    wid = c * NS + s
    pltpu.sync_copy(srcr.at[wid], src_i)
    pltpu.sync_copy(dstr.at[wid], dst_i)
    pltpu.sync_copy(ones_hbm, ones_b)
    pltpu.sync_copy(zeros_hbm.at[pl.ds(s * RPS, RPS)],
                    acco.at[pl.ds(s * RPS, RPS)])
    pltpu.sync_copy(zeros_hbm.at[pl.ds(s * RPS, RPS)],
                    acci.at[pl.ds(s * RPS, RPS)])
    plsc.subcore_barrier()

    for b in range(NBUF):
        pltpu.async_copy(ones_b, acco.at[src_i.at[b]], osem.at[b], add=True)
        pltpu.async_copy(ones_b, acci.at[dst_i.at[b]], isem.at[b], add=True)

    @pl.loop(NBUF, NCH, step=NBUF)
    def _step(j0):
        for b in range(NBUF):
            pltpu.make_async_copy(ones_b, acco.at[src_i.at[0]], osem.at[b]).wait()
            pltpu.make_async_copy(ones_b, acci.at[dst_i.at[0]], isem.at[b]).wait()
            pltpu.async_copy(ones_b, acco.at[src_i.at[j0 + b]], osem.at[b], add=True)
            pltpu.async_copy(ones_b, acci.at[dst_i.at[j0 + b]], isem.at[b], add=True)

    for b in range(NBUF):
        pltpu.make_async_copy(ones_b, acco.at[src_i.at[0]], osem.at[b]).wait()
        pltpu.make_async_copy(ones_b, acci.at[dst_i.at[0]], isem.at[b]).wait()

    plsc.subcore_barrier()
    pltpu.sync_copy(acco.at[pl.ds(s * RPS, RPS)],
                    dego_hbm.at[c, pl.ds(s * RPS, RPS)])
    pltpu.sync_copy(acci.at[pl.ds(s * RPS, RPS)],
                    degi_hbm.at[c, pl.ds(s * RPS, RPS)])


_deg_call = pl.kernel(
    _deg_body,
    out_type=(jax.ShapeDtypeStruct((NC, NPAD, 16), jnp.float32),
              jax.ShapeDtypeStruct((NC, NPAD, 16), jnp.float32)),
    mesh=_MESH,
    compiler_params=_SC_PARAMS,
    scratch_types=[
        pltpu.VMEM((NCH, CHE), jnp.int32),
        pltpu.VMEM((NCH, CHE), jnp.int32),
        pltpu.VMEM((CHE, 16), jnp.float32),
        pltpu.SemaphoreType.DMA((NBUF,)),
        pltpu.SemaphoreType.DMA((NBUF,)),
        pltpu.VMEM_SHARED((NPAD, 16), jnp.float32),
        pltpu.VMEM_SHARED((NPAD, 16), jnp.float32),
    ],
)


_BS = 512
_GRID = NPAD // _BS


def _norm_kernel(dego_ref, degi_ref, nsrc_ref, ndst_ref):
    dego = dego_ref[0, :, 0:1] + dego_ref[1, :, 0:1]
    degi = degi_ref[0, :, 0:1] + degi_ref[1, :, 0:1]
    nsrc_ref[...] = lax.rsqrt(jnp.maximum(dego, 1.0))
    ndst_ref[...] = lax.rsqrt(jnp.maximum(degi, 1.0))


def _norms(dego, degi):
    return pl.pallas_call(
        _norm_kernel,
        out_shape=(jax.ShapeDtypeStruct((NPAD, 1), jnp.float32),
                   jax.ShapeDtypeStruct((NPAD, 1), jnp.float32)),
    )(dego, degi)


def _mm_kernel(x_ref, w_ref, nsrc_ref, h_ref, v_ref):
    h = jnp.dot(x_ref[...], w_ref[...], preferred_element_type=jnp.float32)
    h_ref[...] = h
    v_ref[...] = h * nsrc_ref[...]


def _mm_prescale(x, w, nsrc):
    din = x.shape[1]
    f = w.shape[1]
    return pl.pallas_call(
        _mm_kernel,
        grid=(_GRID,),
        in_specs=[
            pl.BlockSpec((_BS, din), lambda i: (i, 0)),
            pl.BlockSpec((din, f), lambda i: (0, 0)),
            pl.BlockSpec((_BS, 1), lambda i: (i, 0)),
        ],
        out_specs=(pl.BlockSpec((_BS, f), lambda i: (i, 0)),
                   pl.BlockSpec((_BS, f), lambda i: (i, 0))),
        out_shape=(jax.ShapeDtypeStruct((NPAD, f), jnp.float32),
                   jax.ShapeDtypeStruct((NPAD, f), jnp.float32)),
    )(x, w, nsrc)


def _combine_kernel(p_ref, ndst_ref, nsrc_ref, cur_ref, v_ref):
    agg = p_ref[0] + p_ref[1]
    cur = agg * ndst_ref[...]
    cur_ref[...] = cur
    v_ref[...] = cur * nsrc_ref[...]


def _combine(parts, ndst, nsrc):
    f = parts.shape[2]
    return pl.pallas_call(
        _combine_kernel,
        grid=(_GRID,),
        in_specs=[
            pl.BlockSpec((NC, _BS, f), lambda i: (0, i, 0)),
            pl.BlockSpec((_BS, 1), lambda i: (i, 0)),
            pl.BlockSpec((_BS, 1), lambda i: (i, 0)),
        ],
        out_specs=(pl.BlockSpec((_BS, f), lambda i: (i, 0)),
                   pl.BlockSpec((_BS, f), lambda i: (i, 0))),
        out_shape=(jax.ShapeDtypeStruct((NPAD, f), jnp.float32),
                   jax.ShapeDtypeStruct((NPAD, f), jnp.float32)),
    )(parts, ndst, nsrc)


def _attn_kernel(h0_ref, h1_ref, h2_ref, h3_ref, pos_ref, al_ref, ar_ref,
                 b_ref, out_ref):
    hs = [h0_ref[...] + pos_ref[0:1, :],
          h1_ref[...] + pos_ref[1:2, :],
          h2_ref[...] + pos_ref[2:3, :],
          h3_ref[...] + pos_ref[3:4, :]]
    al = al_ref[...]
    ar = ar_ref[...]
    a_r = jnp.sum(hs[0] * ar, axis=1, keepdims=True)
    a = [jnp.sum(h * al, axis=1, keepdims=True) + a_r for h in hs]
    a = [jnp.where(ak >= 0, ak, NEG_SLOPE * ak) for ak in a]
    m = jnp.maximum(jnp.maximum(a[0], a[1]), jnp.maximum(a[2], a[3]))
    e = [jnp.exp(ak - m) for ak in a]
    denom = e[0] + e[1] + e[2] + e[3]
    rst = hs[0] * (e[0] / denom)
    for k2 in range(1, K + 1):
        rst = rst + hs[k2] * (e[k2] / denom)
    out_ref[...] = rst + b_ref[...]


def _attention(h0, h1, h2, h3, pos, al, ar, b):
    f = h0.shape[1]
    blk = lambda i: (i, 0)
    row = lambda i: (0, 0)
    return pl.pallas_call(
        _attn_kernel,
        grid=(_GRID,),
        in_specs=[
            pl.BlockSpec((_BS, f), blk),
            pl.BlockSpec((_BS, f), blk),
            pl.BlockSpec((_BS, f), blk),
            pl.BlockSpec((_BS, f), blk),
            pl.BlockSpec((K + 1, f), row),
            pl.BlockSpec((1, f), row),
            pl.BlockSpec((1, f), row),
            pl.BlockSpec((1, f), row),
        ],
        out_specs=pl.BlockSpec((_BS, f), blk),
        out_shape=jax.ShapeDtypeStruct((NPAD, f), jnp.float32),
    )(h0, h1, h2, h3, pos, al, ar, b)


def _bn_relu_kernel(x_ref, g_ref, be_ref, out_ref):
    x = x_ref[...]
    rows = lax.broadcasted_iota(jnp.int32, (NPAD, 1), 0)
    mask = rows < N_NODES
    xm = jnp.where(mask, x, 0.0)
    mean = jnp.sum(xm, axis=0, keepdims=True) / N_NODES
    d = x - mean
    dm = jnp.where(mask, d, 0.0)
    var = jnp.sum(dm * dm, axis=0, keepdims=True) / N_NODES
    y = d * lax.rsqrt(var + EPS) * g_ref[...] + be_ref[...]
    out_ref[...] = jnp.maximum(y, 0.0)


def _bn_relu(x, g, be):
    return pl.pallas_call(
        _bn_relu_kernel,
        out_shape=jax.ShapeDtypeStruct((NPAD, F), jnp.float32),
    )(x, g.reshape(1, F), be.reshape(1, F))


def _layer(h_in, srcr, dstr, zeros, nsrc, ndst, w, pos, al, ar, b):
    f = w.shape[1]
    h0, v = _mm_prescale(h_in, w, nsrc)
    hstack = [h0]
    for _ in range(K):
        parts = _hop_call[f](v, srcr, dstr, zeros)
        cur, v = _combine(parts, ndst, nsrc)
        hstack.append(cur)
    return _attention(hstack[0], hstack[1], hstack[2], hstack[3],
                      pos, al, ar, b)


@jax.jit
def kernel(x, edge_index, W0, pos0, al0, ar0, b0, g0, be0,
           W1, pos1, al1, ar1, b1, g1, be1, W2, pos2, al2, ar2, b2):
    f32 = jnp.float32
    xp = jnp.zeros((NPAD, x.shape[1]), f32).at[:N_NODES].set(x)
    srcr = edge_index[0].reshape(NW, NCH, CHE)
    dstr = edge_index[1].reshape(NW, NCH, CHE)
    zeros = jnp.zeros((NPAD, F), f32)
    zeros48 = jnp.zeros((NPAD, F2), f32)
    zeros16 = jnp.zeros((NPAD, 16), f32)
    ones16 = jnp.ones((CHE, 16), f32)

    dego, degi = _deg_call(srcr, dstr, ones16, zeros16)
    nsrc, ndst = _norms(dego, degi)

    w2p = jnp.zeros((F, F2), f32).at[:, :40].set(W2)
    pos2p = jnp.zeros((K + 1, F2), f32).at[:, :40].set(pos2.reshape(K + 1, 40))
    al2p = jnp.zeros((1, F2), f32).at[:, :40].set(al2.reshape(1, 40))
    ar2p = jnp.zeros((1, F2), f32).at[:, :40].set(ar2.reshape(1, 40))
    b2p = jnp.zeros((1, F2), f32).at[:, :40].set(b2.reshape(1, 40))

    h = _layer(xp, srcr, dstr, zeros, nsrc, ndst,
               W0, pos0.reshape(K + 1, F), al0.reshape(1, F),
               ar0.reshape(1, F), b0.reshape(1, F))
    h = _bn_relu(h, g0, be0)
    h = _layer(h, srcr, dstr, zeros, nsrc, ndst,
               W1, pos1.reshape(K + 1, F), al1.reshape(1, F),
               ar1.reshape(1, F), b1.reshape(1, F))
    h = _bn_relu(h, g1, be1)
    h = _layer(h, srcr, dstr, zeros48, nsrc, ndst,
               w2p, pos2p, al2p, ar2p, b2p)
    return h[:N_NODES, :40]


# speedup vs baseline: 1.0202x; 1.0024x over previous
import jax
import jax.numpy as jnp
from jax import lax
from jax.experimental import pallas as pl
from jax.experimental.pallas import tpu as pltpu
from jax.experimental.pallas import tpu_sc as plsc

N_NODES = 10000
NPAD = 10240
N_EDGES = 320000
F = 64
F2 = 48
K = 3
NEG_SLOPE = 0.2
EPS = 1e-5

NC = 2
NS = 16
NW = NC * NS
EPW = N_EDGES // NW
CHE = 125
NCH = EPW // CHE
RPS = NPAD // NS
NBUF = 8

_MESH = plsc.VectorSubcoreMesh(core_axis_name="c", subcore_axis_name="s")
_SC_PARAMS = pltpu.CompilerParams(use_tc_tiling_on_sc=False)


def _hop_body(v_hbm, srcr, dstr, zeros_hbm, out_hbm, src_i, dst_i, bufs,
              gsem, ssem, acc):
    c = lax.axis_index("c")
    s = lax.axis_index("s")
    wid = c * NS + s
    cp_src = pltpu.async_copy(srcr.at[wid], src_i, gsem.at[0])
    cp_dst = pltpu.async_copy(dstr.at[wid], dst_i, gsem.at[1])
    cp_z = pltpu.async_copy(zeros_hbm.at[pl.ds(s * RPS, RPS)],
                            acc.at[pl.ds(s * RPS, RPS)], gsem.at[2])
    cp_src.wait()
    cp_dst.wait()
    cp_z.wait()
    plsc.subcore_barrier()

    for b in range(NBUF):
        pltpu.async_copy(v_hbm.at[src_i.at[b]], bufs.at[b], gsem.at[b])

    @pl.loop(0, NCH - NBUF, step=NBUF)
    def _outer(j0):
        for b in range(NBUF):
            pltpu.make_async_copy(v_hbm.at[src_i.at[0]], bufs.at[b],
                                  gsem.at[b]).wait()
            pltpu.async_copy(bufs.at[b], acc.at[dst_i.at[j0 + b]],
                             ssem.at[b], add=True)
        for b in range(NBUF):
            pltpu.make_async_copy(bufs.at[b], acc.at[dst_i.at[0]],
                                  ssem.at[b]).wait()
            pltpu.async_copy(v_hbm.at[src_i.at[j0 + NBUF + b]], bufs.at[b],
                             gsem.at[b])

    for b in range(NBUF):
        pltpu.make_async_copy(v_hbm.at[src_i.at[0]], bufs.at[b],
                              gsem.at[b]).wait()
        pltpu.async_copy(bufs.at[b], acc.at[dst_i.at[NCH - NBUF + b]],
                         ssem.at[b], add=True)
    for b in range(NBUF):
        pltpu.make_async_copy(bufs.at[b], acc.at[dst_i.at[0]],
                              ssem.at[b]).wait()

    plsc.subcore_barrier()
    pltpu.sync_copy(acc.at[pl.ds(s * RPS, RPS)],
                    out_hbm.at[c, pl.ds(s * RPS, RPS)])


def _make_hop(f):
    return pl.kernel(
        _hop_body,
        out_type=jax.ShapeDtypeStruct((NC, NPAD, f), jnp.float32),
        mesh=_MESH,
        compiler_params=_SC_PARAMS,
        scratch_types=[
            pltpu.VMEM((NCH, CHE), jnp.int32),
            pltpu.VMEM((NCH, CHE), jnp.int32),
            pltpu.VMEM((NBUF, CHE, f), jnp.float32),
            pltpu.SemaphoreType.DMA((NBUF,)),
            pltpu.SemaphoreType.DMA((NBUF,)),
            pltpu.VMEM_SHARED((NPAD, f), jnp.float32),
        ],
    )


_hop_call = {f: _make_hop(f) for f in (F, F2)}


def _deg_body(srcr, dstr, ones_hbm, zeros_hbm, dego_hbm, degi_hbm,
              src_i, dst_i, ones_b, osem, isem, acco, acci):
    c = lax.axis_index("c")
    s = lax.axis_index("s")
    wid = c * NS + s
    cps = [pltpu.async_copy(srcr.at[wid], src_i, osem.at[0]),
           pltpu.async_copy(dstr.at[wid], dst_i, osem.at[1]),
           pltpu.async_copy(ones_hbm, ones_b, osem.at[2]),
           pltpu.async_copy(zeros_hbm.at[pl.ds(s * RPS, RPS)],
                            acco.at[pl.ds(s * RPS, RPS)], isem.at[0]),
           pltpu.async_copy(zeros_hbm.at[pl.ds(s * RPS, RPS)],
                            acci.at[pl.ds(s * RPS, RPS)], isem.at[1])]
    for cp in cps:
        cp.wait()
    plsc.subcore_barrier()

    for b in range(NBUF):
        pltpu.async_copy(ones_b, acco.at[src_i.at[b]], osem.at[b], add=True)
        pltpu.async_copy(ones_b, acci.at[dst_i.at[b]], isem.at[b], add=True)

    @pl.loop(NBUF, NCH, step=NBUF)
    def _step(j0):
        for b in range(NBUF):
            pltpu.make_async_copy(ones_b, acco.at[src_i.at[0]], osem.at[b]).wait()
            pltpu.make_async_copy(ones_b, acci.at[dst_i.at[0]], isem.at[b]).wait()
            pltpu.async_copy(ones_b, acco.at[src_i.at[j0 + b]], osem.at[b], add=True)
            pltpu.async_copy(ones_b, acci.at[dst_i.at[j0 + b]], isem.at[b], add=True)

    for b in range(NBUF):
        pltpu.make_async_copy(ones_b, acco.at[src_i.at[0]], osem.at[b]).wait()
        pltpu.make_async_copy(ones_b, acci.at[dst_i.at[0]], isem.at[b]).wait()

    plsc.subcore_barrier()
    pltpu.sync_copy(acco.at[pl.ds(s * RPS, RPS)],
                    dego_hbm.at[c, pl.ds(s * RPS, RPS)])
    pltpu.sync_copy(acci.at[pl.ds(s * RPS, RPS)],
                    degi_hbm.at[c, pl.ds(s * RPS, RPS)])


_deg_call = pl.kernel(
    _deg_body,
    out_type=(jax.ShapeDtypeStruct((NC, NPAD, 16), jnp.float32),
              jax.ShapeDtypeStruct((NC, NPAD, 16), jnp.float32)),
    mesh=_MESH,
    compiler_params=_SC_PARAMS,
    scratch_types=[
        pltpu.VMEM((NCH, CHE), jnp.int32),
        pltpu.VMEM((NCH, CHE), jnp.int32),
        pltpu.VMEM((CHE, 16), jnp.float32),
        pltpu.SemaphoreType.DMA((NBUF,)),
        pltpu.SemaphoreType.DMA((NBUF,)),
        pltpu.VMEM_SHARED((NPAD, 16), jnp.float32),
        pltpu.VMEM_SHARED((NPAD, 16), jnp.float32),
    ],
)


_BS = 512
_GRID = NPAD // _BS


def _norm_kernel(dego_ref, degi_ref, nsrc_ref, ndst_ref):
    dego = dego_ref[0, :, 0:1] + dego_ref[1, :, 0:1]
    degi = degi_ref[0, :, 0:1] + degi_ref[1, :, 0:1]
    nsrc_ref[...] = lax.rsqrt(jnp.maximum(dego, 1.0))
    ndst_ref[...] = lax.rsqrt(jnp.maximum(degi, 1.0))


def _norms(dego, degi):
    return pl.pallas_call(
        _norm_kernel,
        out_shape=(jax.ShapeDtypeStruct((NPAD, 1), jnp.float32),
                   jax.ShapeDtypeStruct((NPAD, 1), jnp.float32)),
    )(dego, degi)


def _mm_kernel(x_ref, w_ref, nsrc_ref, h_ref, v_ref):
    h = jnp.dot(x_ref[...], w_ref[...], preferred_element_type=jnp.float32)
    h_ref[...] = h
    v_ref[...] = h * nsrc_ref[...]


def _mm_prescale(x, w, nsrc):
    din = x.shape[1]
    f = w.shape[1]
    return pl.pallas_call(
        _mm_kernel,
        grid=(_GRID,),
        in_specs=[
            pl.BlockSpec((_BS, din), lambda i: (i, 0)),
            pl.BlockSpec((din, f), lambda i: (0, 0)),
            pl.BlockSpec((_BS, 1), lambda i: (i, 0)),
        ],
        out_specs=(pl.BlockSpec((_BS, f), lambda i: (i, 0)),
                   pl.BlockSpec((_BS, f), lambda i: (i, 0))),
        out_shape=(jax.ShapeDtypeStruct((NPAD, f), jnp.float32),
                   jax.ShapeDtypeStruct((NPAD, f), jnp.float32)),
    )(x, w, nsrc)


def _combine_kernel(p_ref, ndst_ref, nsrc_ref, cur_ref, v_ref):
    agg = p_ref[0] + p_ref[1]
    cur = agg * ndst_ref[...]
    cur_ref[...] = cur
    v_ref[...] = cur * nsrc_ref[...]


def _combine(parts, ndst, nsrc):
    f = parts.shape[2]
    return pl.pallas_call(
        _combine_kernel,
        grid=(_GRID,),
        in_specs=[
            pl.BlockSpec((NC, _BS, f), lambda i: (0, i, 0)),
            pl.BlockSpec((_BS, 1), lambda i: (i, 0)),
            pl.BlockSpec((_BS, 1), lambda i: (i, 0)),
        ],
        out_specs=(pl.BlockSpec((_BS, f), lambda i: (i, 0)),
                   pl.BlockSpec((_BS, f), lambda i: (i, 0))),
        out_shape=(jax.ShapeDtypeStruct((NPAD, f), jnp.float32),
                   jax.ShapeDtypeStruct((NPAD, f), jnp.float32)),
    )(parts, ndst, nsrc)


def _attn_kernel(h0_ref, h1_ref, h2_ref, h3_ref, pos_ref, al_ref, ar_ref,
                 b_ref, out_ref):
    hs = [h0_ref[...] + pos_ref[0:1, :],
          h1_ref[...] + pos_ref[1:2, :],
          h2_ref[...] + pos_ref[2:3, :],
          h3_ref[...] + pos_ref[3:4, :]]
    al = al_ref[...]
    ar = ar_ref[...]
    a_r = jnp.sum(hs[0] * ar, axis=1, keepdims=True)
    a = [jnp.sum(h * al, axis=1, keepdims=True) + a_r for h in hs]
    a = [jnp.where(ak >= 0, ak, NEG_SLOPE * ak) for ak in a]
    m = jnp.maximum(jnp.maximum(a[0], a[1]), jnp.maximum(a[2], a[3]))
    e = [jnp.exp(ak - m) for ak in a]
    denom = e[0] + e[1] + e[2] + e[3]
    rst = hs[0] * (e[0] / denom)
    for k2 in range(1, K + 1):
        rst = rst + hs[k2] * (e[k2] / denom)
    out_ref[...] = rst + b_ref[...]


def _attention(h0, h1, h2, h3, pos, al, ar, b):
    f = h0.shape[1]
    blk = lambda i: (i, 0)
    row = lambda i: (0, 0)
    return pl.pallas_call(
        _attn_kernel,
        grid=(_GRID,),
        in_specs=[
            pl.BlockSpec((_BS, f), blk),
            pl.BlockSpec((_BS, f), blk),
            pl.BlockSpec((_BS, f), blk),
            pl.BlockSpec((_BS, f), blk),
            pl.BlockSpec((K + 1, f), row),
            pl.BlockSpec((1, f), row),
            pl.BlockSpec((1, f), row),
            pl.BlockSpec((1, f), row),
        ],
        out_specs=pl.BlockSpec((_BS, f), blk),
        out_shape=jax.ShapeDtypeStruct((NPAD, f), jnp.float32),
    )(h0, h1, h2, h3, pos, al, ar, b)


def _bn_relu_kernel(x_ref, g_ref, be_ref, out_ref):
    x = x_ref[...]
    rows = lax.broadcasted_iota(jnp.int32, (NPAD, 1), 0)
    mask = rows < N_NODES
    xm = jnp.where(mask, x, 0.0)
    mean = jnp.sum(xm, axis=0, keepdims=True) / N_NODES
    d = x - mean
    dm = jnp.where(mask, d, 0.0)
    var = jnp.sum(dm * dm, axis=0, keepdims=True) / N_NODES
    y = d * lax.rsqrt(var + EPS) * g_ref[...] + be_ref[...]
    out_ref[...] = jnp.maximum(y, 0.0)


def _bn_relu(x, g, be):
    return pl.pallas_call(
        _bn_relu_kernel,
        out_shape=jax.ShapeDtypeStruct((NPAD, F), jnp.float32),
    )(x, g.reshape(1, F), be.reshape(1, F))


def _layer(h_in, srcr, dstr, zeros, nsrc, ndst, w, pos, al, ar, b):
    f = w.shape[1]
    h0, v = _mm_prescale(h_in, w, nsrc)
    hstack = [h0]
    for _ in range(K):
        parts = _hop_call[f](v, srcr, dstr, zeros)
        cur, v = _combine(parts, ndst, nsrc)
        hstack.append(cur)
    return _attention(hstack[0], hstack[1], hstack[2], hstack[3],
                      pos, al, ar, b)


@jax.jit
def kernel(x, edge_index, W0, pos0, al0, ar0, b0, g0, be0,
           W1, pos1, al1, ar1, b1, g1, be1, W2, pos2, al2, ar2, b2):
    f32 = jnp.float32
    xp = jnp.zeros((NPAD, x.shape[1]), f32).at[:N_NODES].set(x)
    srcr = edge_index[0].reshape(NW, NCH, CHE)
    dstr = edge_index[1].reshape(NW, NCH, CHE)
    zeros = jnp.zeros((NPAD, F), f32)
    zeros48 = jnp.zeros((NPAD, F2), f32)
    zeros16 = jnp.zeros((NPAD, 16), f32)
    ones16 = jnp.ones((CHE, 16), f32)

    dego, degi = _deg_call(srcr, dstr, ones16, zeros16)
    nsrc, ndst = _norms(dego, degi)

    w2p = jnp.zeros((F, F2), f32).at[:, :40].set(W2)
    pos2p = jnp.zeros((K + 1, F2), f32).at[:, :40].set(pos2.reshape(K + 1, 40))
    al2p = jnp.zeros((1, F2), f32).at[:, :40].set(al2.reshape(1, 40))
    ar2p = jnp.zeros((1, F2), f32).at[:, :40].set(ar2.reshape(1, 40))
    b2p = jnp.zeros((1, F2), f32).at[:, :40].set(b2.reshape(1, 40))

    h = _layer(xp, srcr, dstr, zeros, nsrc, ndst,
               W0, pos0.reshape(K + 1, F), al0.reshape(1, F),
               ar0.reshape(1, F), b0.reshape(1, F))
    h = _bn_relu(h, g0, be0)
    h = _layer(h, srcr, dstr, zeros, nsrc, ndst,
               W1, pos1.reshape(K + 1, F), al1.reshape(1, F),
               ar1.reshape(1, F), b1.reshape(1, F))
    h = _bn_relu(h, g1, be1)
    h = _layer(h, srcr, dstr, zeros48, nsrc, ndst,
               w2p, pos2p, al2p, ar2p, b2p)
    return h[:N_NODES, :40]
